# baseline (device time: 20525 ns/iter reference)
import jax
import jax.numpy as jnp
from jax import lax
from jax.experimental import pallas as pl
from jax.experimental.pallas import tpu as pltpu

N_TOK = 512
D = 512
F = 1024
E_LOCAL = 2
CHUNKS = 4
H = N_TOK // CHUNKS
LOCAL_PRE = 2
W = D + 128


def kernel(x, assign, W1, W2):
    assign_col = assign.reshape(N_TOK, 1)
    x = pltpu.with_memory_space_constraint(x, pltpu.MemorySpace.HBM)
    assign_col = pltpu.with_memory_space_constraint(
        assign_col, pltpu.MemorySpace.HBM)
    W1 = pltpu.with_memory_space_constraint(W1, pltpu.MemorySpace.HBM)
    W2 = pltpu.with_memory_space_constraint(W2, pltpu.MemorySpace.HBM)

    def body(x_ref, a_ref, w1_ref, w2_ref, out_ref,
             xv_ref, av_ref, w1v_ref, w2v_ref, xb_ref, recv_x_ref,
             send_o_ref, recv_o_ref, outv_ref,
             send_sems, recv_sems, ldma_sems, odma_sems):
        my_x = lax.axis_index("x")
        my_y = lax.axis_index("y")
        my_z = lax.axis_index("z")
        peer = (1 - my_x, my_y, my_z)

        def mrc(src, dst, i):
            return pltpu.make_async_remote_copy(
                src_ref=src, dst_ref=dst,
                send_sem=send_sems.at[i], recv_sem=recv_sems.at[i],
                device_id=peer, device_id_type=pl.DeviceIdType.MESH)

        x_dma = pltpu.make_async_copy(x_ref, xv_ref, ldma_sems.at[0])
        x_dma.start()
        a_dma = pltpu.make_async_copy(a_ref, av_ref, ldma_sems.at[1])
        a_dma.start()
        w1_dma = pltpu.make_async_copy(w1_ref, w1v_ref, ldma_sems.at[2])
        w1_dma.start()
        w2_dma = pltpu.make_async_copy(w2_ref, w2v_ref, ldma_sems.at[3])
        w2_dma.start()

        barrier = pltpu.get_barrier_semaphore()
        pl.semaphore_signal(barrier, inc=1, device_id=peer,
                            device_id_type=pl.DeviceIdType.MESH)
        pl.semaphore_wait(barrier, 1)

        x_dma.wait()
        a_dma.wait()
        xb_ref[:, D:D + 1] = av_ref[...].astype(jnp.bfloat16)
        rdma_x = []
        for c in range(CHUNKS):
            sl = pl.ds(c * H, H)
            xb_ref[sl, :D] = xv_ref[sl, :].astype(jnp.bfloat16)
            r = mrc(xb_ref.at[sl], recv_x_ref.at[sl], c)
            r.start()
            rdma_x.append(r)

        w1_dma.wait()
        w2_dma.wait()
        w1b = [w1v_ref[e].astype(jnp.bfloat16) for e in range(E_LOCAL)]
        w2b = [w2v_ref[e].astype(jnp.bfloat16) for e in range(E_LOCAL)]

        def ffn(tok, a_col):
            acc = jnp.zeros(tok.shape, jnp.float32)
            for e in range(E_LOCAL):
                ge = (my_x * E_LOCAL + e).astype(a_col.dtype)
                xm = jnp.where(a_col == ge, tok, 0)
                h = jnp.dot(xm, w1b[e], preferred_element_type=jnp.float32)
                h = jnp.maximum(h, 0.0).astype(jnp.bfloat16)
                acc = acc + jnp.dot(h, w2b[e],
                                    preferred_element_type=jnp.float32)
            return acc

        def local_chunk(c):
            sl = pl.ds(c * H, H)
            return ffn(xb_ref[sl, :D], av_ref[sl, :])

        acc_parts = {}
        for c in range(LOCAL_PRE):
            acc_parts[c] = local_chunk(c)

        rdma_o = []
        for c in range(CHUNKS):
            sl = pl.ds(c * H, H)
            rdma_x[c].wait_recv()
            acc = ffn(recv_x_ref[sl, :D], recv_x_ref[sl, D:D + 1])
            send_o_ref[sl, :] = acc.astype(jnp.bfloat16)
            r = mrc(send_o_ref.at[sl], recv_o_ref.at[sl], CHUNKS + c)
            r.start()
            rdma_o.append(r)

        for c in range(LOCAL_PRE, CHUNKS):
            acc_parts[c] = local_chunk(c)

        out_dmas = []
        for c in range(CHUNKS):
            sl = pl.ds(c * H, H)
            rdma_o[c].wait_recv()
            outv_ref[sl, :] = acc_parts[c] + recv_o_ref[sl, :].astype(
                jnp.float32)
            od = pltpu.make_async_copy(
                outv_ref.at[sl], out_ref.at[sl], odma_sems.at[c])
            od.start()
            out_dmas.append(od)

        for od in out_dmas:
            od.wait()
        for r in rdma_x:
            r.wait_send()
        for r in rdma_o:
            r.wait_send()

    n_sems = 2 * CHUNKS
    return pl.pallas_call(
        body,
        out_shape=jax.ShapeDtypeStruct((N_TOK, D), jnp.float32),
        in_specs=[
            pl.BlockSpec(memory_space=pltpu.MemorySpace.HBM),
            pl.BlockSpec(memory_space=pltpu.MemorySpace.HBM),
            pl.BlockSpec(memory_space=pltpu.MemorySpace.HBM),
            pl.BlockSpec(memory_space=pltpu.MemorySpace.HBM),
        ],
        out_specs=pl.BlockSpec(memory_space=pltpu.MemorySpace.HBM),
        scratch_shapes=[
            pltpu.VMEM((N_TOK, D), jnp.float32),
            pltpu.VMEM((N_TOK, 1), jnp.int32),
            pltpu.VMEM((E_LOCAL, D, F), jnp.float32),
            pltpu.VMEM((E_LOCAL, F, D), jnp.float32),
            pltpu.VMEM((N_TOK, W), jnp.bfloat16),
            pltpu.VMEM((N_TOK, W), jnp.bfloat16),
            pltpu.VMEM((N_TOK, D), jnp.bfloat16),
            pltpu.VMEM((N_TOK, D), jnp.bfloat16),
            pltpu.VMEM((N_TOK, D), jnp.float32),
            pltpu.SemaphoreType.DMA((n_sems,)),
            pltpu.SemaphoreType.DMA((n_sems,)),
            pltpu.SemaphoreType.DMA((4,)),
            pltpu.SemaphoreType.DMA((CHUNKS,)),
        ],
        compiler_params=pltpu.CompilerParams(collective_id=0),
    )(x, assign_col, W1, W2)


# device time: 19597 ns/iter; 1.0474x vs baseline; 1.0474x over previous
import jax
import jax.numpy as jnp
from jax import lax
from jax.experimental import pallas as pl
from jax.experimental.pallas import tpu as pltpu

N_TOK = 512
D = 512
F = 1024
E_LOCAL = 2
CHUNKS = 4
H = N_TOK // CHUNKS
LOCAL_PRE = 2
W = D + 128


def kernel(x, assign, W1, W2):
    assign_col = assign.reshape(N_TOK, 1)
    W1 = pltpu.with_memory_space_constraint(W1, pltpu.MemorySpace.HBM)
    W2 = pltpu.with_memory_space_constraint(W2, pltpu.MemorySpace.HBM)

    def body(x_ref, a_ref, w1_ref, w2_ref, out_ref,
             w1v_ref, w2v_ref, xb_ref, recv_x_ref, send_o_ref,
             recv_o_ref, outv_ref, send_sems, recv_sems, ldma_sems,
             odma_sems):
        my_x = lax.axis_index("x")
        my_y = lax.axis_index("y")
        my_z = lax.axis_index("z")
        peer = (1 - my_x, my_y, my_z)

        def mrc(src, dst, i):
            return pltpu.make_async_remote_copy(
                src_ref=src, dst_ref=dst,
                send_sem=send_sems.at[i], recv_sem=recv_sems.at[i],
                device_id=peer, device_id_type=pl.DeviceIdType.MESH)

        w1_dma = pltpu.make_async_copy(w1_ref, w1v_ref, ldma_sems.at[0])
        w1_dma.start()
        w2_dma = pltpu.make_async_copy(w2_ref, w2v_ref, ldma_sems.at[1])
        w2_dma.start()

        barrier = pltpu.get_barrier_semaphore()
        pl.semaphore_signal(barrier, inc=1, device_id=peer,
                            device_id_type=pl.DeviceIdType.MESH)
        pl.semaphore_wait(barrier, 1)

        xb_ref[:, D:D + 1] = a_ref[...].astype(jnp.bfloat16)
        rdma_x = []
        for c in range(CHUNKS):
            sl = pl.ds(c * H, H)
            xb_ref[sl, :D] = x_ref[sl, :].astype(jnp.bfloat16)
            r = mrc(xb_ref.at[sl], recv_x_ref.at[sl], c)
            r.start()
            rdma_x.append(r)

        w1_dma.wait()
        w2_dma.wait()
        w1b = [w1v_ref[e].astype(jnp.bfloat16) for e in range(E_LOCAL)]
        w2b = [w2v_ref[e].astype(jnp.bfloat16) for e in range(E_LOCAL)]

        def ffn(tok, a_col):
            acc = jnp.zeros(tok.shape, jnp.float32)
            for e in range(E_LOCAL):
                ge = (my_x * E_LOCAL + e).astype(a_col.dtype)
                xm = jnp.where(a_col == ge, tok, 0)
                h = jnp.dot(xm, w1b[e], preferred_element_type=jnp.float32)
                h = jnp.maximum(h, 0.0).astype(jnp.bfloat16)
                acc = acc + jnp.dot(h, w2b[e],
                                    preferred_element_type=jnp.float32)
            return acc

        def local_chunk(c):
            sl = pl.ds(c * H, H)
            return ffn(xb_ref[sl, :D], a_ref[sl, :])

        acc_parts = {}
        for c in range(LOCAL_PRE):
            acc_parts[c] = local_chunk(c)

        rdma_o = []
        for c in range(CHUNKS):
            sl = pl.ds(c * H, H)
            rdma_x[c].wait_recv()
            acc = ffn(recv_x_ref[sl, :D], recv_x_ref[sl, D:D + 1])
            send_o_ref[sl, :] = acc.astype(jnp.bfloat16)
            r = mrc(send_o_ref.at[sl], recv_o_ref.at[sl], CHUNKS + c)
            r.start()
            rdma_o.append(r)

        for c in range(LOCAL_PRE, CHUNKS):
            acc_parts[c] = local_chunk(c)

        out_dmas = []
        for c in range(CHUNKS):
            sl = pl.ds(c * H, H)
            rdma_o[c].wait_recv()
            outv_ref[sl, :] = acc_parts[c] + recv_o_ref[sl, :].astype(
                jnp.float32)
            od = pltpu.make_async_copy(
                outv_ref.at[sl], out_ref.at[sl], odma_sems.at[c])
            od.start()
            out_dmas.append(od)

        for od in out_dmas:
            od.wait()
        for r in rdma_x:
            r.wait_send()
        for r in rdma_o:
            r.wait_send()

    n_sems = 2 * CHUNKS
    return pl.pallas_call(
        body,
        out_shape=jax.ShapeDtypeStruct((N_TOK, D), jnp.float32),
        in_specs=[
            pl.BlockSpec(memory_space=pltpu.VMEM),
            pl.BlockSpec(memory_space=pltpu.VMEM),
            pl.BlockSpec(memory_space=pltpu.MemorySpace.HBM),
            pl.BlockSpec(memory_space=pltpu.MemorySpace.HBM),
        ],
        out_specs=pl.BlockSpec(memory_space=pltpu.MemorySpace.HBM),
        scratch_shapes=[
            pltpu.VMEM((E_LOCAL, D, F), jnp.float32),
            pltpu.VMEM((E_LOCAL, F, D), jnp.float32),
            pltpu.VMEM((N_TOK, W), jnp.bfloat16),
            pltpu.VMEM((N_TOK, W), jnp.bfloat16),
            pltpu.VMEM((N_TOK, D), jnp.bfloat16),
            pltpu.VMEM((N_TOK, D), jnp.bfloat16),
            pltpu.VMEM((N_TOK, D), jnp.float32),
            pltpu.SemaphoreType.DMA((n_sems,)),
            pltpu.SemaphoreType.DMA((n_sems,)),
            pltpu.SemaphoreType.DMA((2,)),
            pltpu.SemaphoreType.DMA((CHUNKS,)),
        ],
        compiler_params=pltpu.CompilerParams(collective_id=0),
    )(x, assign_col, W1, W2)


# device time: 17497 ns/iter; 1.1731x vs baseline; 1.1200x over previous
import functools

import jax
import jax.numpy as jnp
from jax import lax
from jax.experimental import pallas as pl
from jax.experimental.pallas import tpu as pltpu

N_TOK = 512
D = 512
F = 1024
E_LOCAL = 2
HALF = N_TOK // 2
CHUNKS = 2
H = HALF // CHUNKS
LOCAL_PRE = 1
W = D + 128


def kernel(x, assign, W1, W2):
    assign_col = assign.reshape(N_TOK, 1)
    x = pltpu.with_memory_space_constraint(x, pltpu.MemorySpace.HBM)
    W1 = pltpu.with_memory_space_constraint(W1, pltpu.MemorySpace.HBM)
    W2 = pltpu.with_memory_space_constraint(W2, pltpu.MemorySpace.HBM)

    def body(x_ref, a_ref, w1_ref, w2_ref, out_ref,
             w1v_ref, w2v_ref, xv_ref, xb_ref, recv_x_ref, send_o_ref,
             recv_o_ref, swap_ref, recv_swap_ref,
             send_sems, recv_sems, ldma_sems, buddy_sem):
        my_x = lax.axis_index("x")
        my_y = lax.axis_index("y")
        my_z = lax.axis_index("z")
        peer = (1 - my_x, my_y, my_z)
        buddy = (my_x, my_y, my_z ^ 1)
        h = my_z % 2
        base = h * HALF
        obase = (1 - h) * HALF

        def mrc(src, dst, i, dev):
            return pltpu.make_async_remote_copy(
                src_ref=src, dst_ref=dst,
                send_sem=send_sems.at[i], recv_sem=recv_sems.at[i],
                device_id=dev, device_id_type=pl.DeviceIdType.MESH)

        x_dmas = []
        for c in range(CHUNKS):
            sl = pl.ds(c * H, H)
            xd = pltpu.make_async_copy(
                x_ref.at[pl.ds(base + c * H, H)], xv_ref.at[sl],
                ldma_sems.at[2 + c])
            xd.start()
            x_dmas.append(xd)
        w1_dma = pltpu.make_async_copy(w1_ref, w1v_ref, ldma_sems.at[0])
        w1_dma.start()
        w2_dma = pltpu.make_async_copy(w2_ref, w2v_ref, ldma_sems.at[1])
        w2_dma.start()

        barrier = pltpu.get_barrier_semaphore()
        pl.semaphore_signal(barrier, inc=1, device_id=peer,
                            device_id_type=pl.DeviceIdType.MESH)
        pl.semaphore_signal(buddy_sem, inc=1, device_id=buddy,
                            device_id_type=pl.DeviceIdType.MESH)
        pl.semaphore_wait(barrier, 1)

        xb_ref[:, D:D + 1] = a_ref[pl.ds(base, HALF), :].astype(
            jnp.bfloat16)
        rdma_x = []
        for c in range(CHUNKS):
            sl = pl.ds(c * H, H)
            x_dmas[c].wait()
            xb_ref[sl, :D] = xv_ref[sl, :].astype(jnp.bfloat16)
            r = mrc(xb_ref.at[sl], recv_x_ref.at[sl], c, peer)
            r.start()
            rdma_x.append(r)

        w1_dma.wait()
        w2_dma.wait()
        w1b = [w1v_ref[e].astype(jnp.bfloat16) for e in range(E_LOCAL)]
        w2b = [w2v_ref[e].astype(jnp.bfloat16) for e in range(E_LOCAL)]

        def ffn(tok, a_col):
            acc = jnp.zeros(tok.shape, jnp.float32)
            for e in range(E_LOCAL):
                ge = (my_x * E_LOCAL + e).astype(a_col.dtype)
                xm = jnp.where(a_col == ge, tok, 0)
                h1 = jnp.dot(xm, w1b[e], preferred_element_type=jnp.float32)
                h1 = jnp.maximum(h1, 0.0).astype(jnp.bfloat16)
                acc = acc + jnp.dot(h1, w2b[e],
                                    preferred_element_type=jnp.float32)
            return acc

        def local_chunk(c):
            sl = pl.ds(c * H, H)
            return ffn(xb_ref[sl, :D], xb_ref[sl, D:D + 1])

        acc_parts = {}
        for c in range(LOCAL_PRE):
            acc_parts[c] = local_chunk(c)

        rdma_o = []
        for c in range(CHUNKS):
            sl = pl.ds(c * H, H)
            rdma_x[c].wait_recv()
            acc = ffn(recv_x_ref[sl, :D], recv_x_ref[sl, D:D + 1])
            send_o_ref[sl, :] = acc.astype(jnp.bfloat16)
            r = mrc(send_o_ref.at[sl], recv_o_ref.at[sl], CHUNKS + c, peer)
            r.start()
            rdma_o.append(r)

        for c in range(LOCAL_PRE, CHUNKS):
            acc_parts[c] = local_chunk(c)

        pl.semaphore_wait(buddy_sem, 1)
        rdma_s = []
        for c in range(CHUNKS):
            sl = pl.ds(c * H, H)
            rdma_o[c].wait_recv()
            full = acc_parts[c] + recv_o_ref[sl, :].astype(jnp.float32)
            out_ref[pl.ds(base + c * H, H), :] = full
            swap_ref[sl, :] = full.astype(jnp.bfloat16)
            r = mrc(swap_ref.at[sl], recv_swap_ref.at[sl],
                    2 * CHUNKS + c, buddy)
            r.start()
            rdma_s.append(r)

        for c in range(CHUNKS):
            sl = pl.ds(c * H, H)
            rdma_s[c].wait_recv()
            out_ref[pl.ds(obase + c * H, H), :] = recv_swap_ref[
                sl, :].astype(jnp.float32)

        for r in rdma_x + rdma_o + rdma_s:
            r.wait_send()

    n_sems = 3 * CHUNKS
    return pl.pallas_call(
        body,
        out_shape=jax.ShapeDtypeStruct((N_TOK, D), jnp.float32),
        in_specs=[
            pl.BlockSpec(memory_space=pltpu.MemorySpace.HBM),
            pl.BlockSpec(memory_space=pltpu.VMEM),
            pl.BlockSpec(memory_space=pltpu.MemorySpace.HBM),
            pl.BlockSpec(memory_space=pltpu.MemorySpace.HBM),
        ],
        out_specs=pl.BlockSpec(memory_space=pltpu.VMEM),
        scratch_shapes=[
            pltpu.VMEM((E_LOCAL, D, F), jnp.float32),
            pltpu.VMEM((E_LOCAL, F, D), jnp.float32),
            pltpu.VMEM((HALF, D), jnp.float32),
            pltpu.VMEM((HALF, W), jnp.bfloat16),
            pltpu.VMEM((HALF, W), jnp.bfloat16),
            pltpu.VMEM((HALF, D), jnp.bfloat16),
            pltpu.VMEM((HALF, D), jnp.bfloat16),
            pltpu.VMEM((HALF, D), jnp.bfloat16),
            pltpu.VMEM((HALF, D), jnp.bfloat16),
            pltpu.SemaphoreType.DMA((n_sems,)),
            pltpu.SemaphoreType.DMA((n_sems,)),
            pltpu.SemaphoreType.DMA((2 + CHUNKS,)),
            pltpu.SemaphoreType.REGULAR,
        ],
        compiler_params=pltpu.CompilerParams(collective_id=0),
    )(x, assign_col, W1, W2)
